# Initial kernel scaffold; baseline (speedup 1.0000x reference)
import jax
import jax.numpy as jnp
from jax.experimental import pallas as pl


def kernel(x, edge_index, edge_attr, pre_w, pre_b, c1_lin, c1_w1, c1_b1, c1_w2, c1_b2, c2_lin, c2_w1, c2_b1, c2_w2, c2_b2, c3_lin, c3_w1, c3_b1, c3_w2, c3_b2, c4_lin, c4_w1, c4_b1, c4_w2, c4_b2, n1_w, n1_b, g2_w, g2_b, g2_ms, g3_w, g3_b, g3_ms, g4_w, g4_b, g4_ms):
    raise NotImplementedError("write your pallas kernel here")


# speedup vs baseline: 13.9565x; 13.9565x over previous
import functools

import jax
import jax.numpy as jnp
from jax import lax
from jax.experimental import pallas as pl
from jax.experimental.pallas import tpu as pltpu
from jax.experimental.pallas import tpu_sc as plsc

N = 10000
E = 320000
D_EDGE = 16
D_HID = 32
DIMS = [128, 104, 80, 56, 32]
PADS = [112, 80, 64, 32]
EPS = 1e-5

NW = 32
ET = E // NW
NP = 10240
NPT = NP // 16


def _edge_mlp_body(ea_ref, w1_ref, b1_ref, w2_ref, b2_ref, out_ref):
    ea = ea_ref[...]
    cols = []
    for l in range(4):
        h = jnp.dot(ea, w1_ref[l], preferred_element_type=jnp.float32)
        h = jnp.maximum(h + b1_ref[l][None, :], 0.0)
        e = jnp.dot(h, w2_ref[l], preferred_element_type=jnp.float32)
        cols.append(jax.nn.sigmoid(e + b2_ref[l][None, :]))
    out_ref[...] = jnp.concatenate(cols, axis=1)


def _edge_mlp(edge_attr, w1s, b1s, w2s, b2s):
    B = 8000
    nblk = E // B
    return pl.pallas_call(
        _edge_mlp_body,
        grid=(nblk,),
        in_specs=[
            pl.BlockSpec((B, D_EDGE), lambda i: (i, 0)),
            pl.BlockSpec((4, D_EDGE, D_HID), lambda i: (0, 0, 0)),
            pl.BlockSpec((4, D_HID), lambda i: (0, 0)),
            pl.BlockSpec((4, D_HID, 1), lambda i: (0, 0, 0)),
            pl.BlockSpec((4, 1), lambda i: (0, 0)),
        ],
        out_specs=pl.BlockSpec((B, 4), lambda i: (i, 0)),
        out_shape=jax.ShapeDtypeStruct((E, 4), jnp.float32),
    )(edge_attr, w1s, b1s, w2s, b2s)


def _pre1_body(x_ref, pw_ref, pb_ref, lin_ref, dega_ref, degb_ref,
               hp_ref, dis_ref):
    x = x_ref[...]
    mean = jnp.mean(x, axis=0)
    var = jnp.mean((x - mean[None, :]) ** 2, axis=0)
    xb = (x - mean[None, :]) * lax.rsqrt(var + EPS)[None, :]
    xb = xb * pw_ref[...][None, :] + pb_ref[...][None, :]
    deg = 1.0 + dega_ref[...][0:N] + degb_ref[...][0:N]
    dis = lax.rsqrt(deg)
    hp = jnp.dot(xb, lin_ref[...], preferred_element_type=jnp.float32)
    hp_ref[...] = hp * dis[:, None]
    dis_ref[...] = dis


def _pre1(x, pre_w, pre_b, lin_pad, dega, degb):
    P = PADS[0]
    return pl.pallas_call(
        _pre1_body,
        out_shape=[
            jax.ShapeDtypeStruct((N, P), jnp.float32),
            jax.ShapeDtypeStruct((N,), jnp.float32),
        ],
    )(x, pre_w, pre_b, lin_pad, dega, degb)


def _mid_body(is_batch_norm, agg0_ref, agg1_ref, hp_ref, dis_ref,
              w_ref, b_ref, ms_ref, lin_ref, dega_ref, degb_ref,
              hp2_ref, dis2_ref):
    agg = agg0_ref[...][0:N] + agg1_ref[...][0:N]
    t = dis_ref[...][:, None] * (agg + hp_ref[...])
    t = jnp.maximum(t, 0.0)
    mean = jnp.mean(t, axis=0)
    if is_batch_norm:
        var = jnp.mean((t - mean[None, :]) ** 2, axis=0)
        t = (t - mean[None, :]) * lax.rsqrt(var + EPS)[None, :]
        t = t * w_ref[...][None, :] + b_ref[...][None, :]
    else:
        out = t - ms_ref[...][None, :] * mean[None, :]
        var = jnp.mean(out * out, axis=0)
        t = (w_ref[...][None, :] * out * lax.rsqrt(var + EPS)[None, :]
             + b_ref[...][None, :])
    deg = 1.0 + dega_ref[...][0:N] + degb_ref[...][0:N]
    dis2 = lax.rsqrt(deg)
    hp2 = jnp.dot(t, lin_ref[...], preferred_element_type=jnp.float32)
    hp2_ref[...] = hp2 * dis2[:, None]
    dis2_ref[...] = dis2


def _mid(l, agg0, agg1, hp, dis, w, b, ms, lin_pad, dega, degb):
    P2 = PADS[l + 1]
    return pl.pallas_call(
        functools.partial(_mid_body, l == 0),
        out_shape=[
            jax.ShapeDtypeStruct((N, P2), jnp.float32),
            jax.ShapeDtypeStruct((N,), jnp.float32),
        ],
    )(agg0, agg1, hp, dis, w, b, ms, lin_pad, dega, degb)


def _final_body(agg0_ref, agg1_ref, hp_ref, dis_ref, w_ref, b_ref, ms_ref,
                out_ref):
    agg = agg0_ref[...][0:N] + agg1_ref[...][0:N]
    t = dis_ref[...][:, None] * (agg + hp_ref[...])
    t = jnp.maximum(t, 0.0)
    mean = jnp.mean(t, axis=0)
    out = t - ms_ref[...][None, :] * mean[None, :]
    var = jnp.mean(out * out, axis=0)
    out_ref[...] = (w_ref[...][None, :] * out * lax.rsqrt(var + EPS)[None, :]
                    + b_ref[...][None, :])


def _final(agg0, agg1, hp, dis, w, b, ms):
    return pl.pallas_call(
        _final_body,
        out_shape=jax.ShapeDtypeStruct((N, DIMS[4]), jnp.float32),
    )(agg0, agg1, hp, dis, w, b, ms)


_MESH = plsc.VectorSubcoreMesh(core_axis_name="c", subcore_axis_name="s")
_SC_PARAMS = pltpu.CompilerParams(needs_layout_passes=False,
                                  use_tc_tiling_on_sc=False)

_DEG_K = 1000


@functools.partial(
    pl.kernel,
    out_type=[jax.ShapeDtypeStruct((NP,), jnp.float32) for _ in range(8)],
    mesh=_MESH,
    compiler_params=_SC_PARAMS,
    scratch_types=[
        pltpu.VMEM((_DEG_K,), jnp.int32),
        pltpu.VMEM((_DEG_K,), jnp.float32),
        pltpu.VMEM((NPT,), jnp.float32),
        pltpu.VMEM_SHARED((NP,), jnp.float32),
        pltpu.VMEM_SHARED((NP,), jnp.float32),
        pltpu.VMEM_SHARED((NP,), jnp.float32),
        pltpu.VMEM_SHARED((NP,), jnp.float32),
    ],
)
def _deg_kernel(col_hbm, ew0_hbm, ew1_hbm, ew2_hbm, ew3_hbm,
                o00, o01, o02, o03, o10, o11, o12, o13,
                col_v, ew_v, zero_v, deg0, deg1, deg2, deg3):
    c = lax.axis_index("c")
    s = lax.axis_index("s")
    wid = c * 16 + s
    degs = [deg0, deg1, deg2, deg3]
    ew_hbms = [ew0_hbm, ew1_hbm, ew2_hbm, ew3_hbm]

    zvec = jnp.zeros((16,), jnp.float32)

    def zbody(i, _):
        zero_v[pl.ds(i * 16, 16)] = zvec
        return 0

    lax.fori_loop(0, NPT // 16, zbody, 0)
    rs = s * NPT
    for d in degs:
        pltpu.sync_copy(zero_v, d.at[pl.ds(rs, NPT)])
    plsc.subcore_barrier()

    def chunk(ch, _):
        base = wid * ET + ch * _DEG_K
        pltpu.sync_copy(col_hbm.at[pl.ds(base, _DEG_K)], col_v)
        for l, d in enumerate(degs):
            pltpu.sync_copy(ew_hbms[l].at[pl.ds(base, _DEG_K)], ew_v)
            pltpu.sync_copy(ew_v, d.at[col_v], add=True)
        return 0

    lax.fori_loop(0, ET // _DEG_K, chunk, 0)
    plsc.subcore_barrier()

    @pl.when(c == 0)
    def _():
        for d, o in zip(degs, [o00, o01, o02, o03]):
            pltpu.sync_copy(d.at[pl.ds(rs, NPT)], o.at[pl.ds(rs, NPT)])

    @pl.when(c == 1)
    def _():
        for d, o in zip(degs, [o10, o11, o12, o13]):
            pltpu.sync_copy(d.at[pl.ds(rs, NPT)], o.at[pl.ds(rs, NPT)])


_MSG_K = 400


def _make_msg_kernel(P):
    @functools.partial(
        pl.kernel,
        out_type=[jax.ShapeDtypeStruct((NP, P), jnp.float32) for _ in range(2)],
        mesh=_MESH,
        compiler_params=_SC_PARAMS,
        scratch_types=[
            pltpu.VMEM((_MSG_K,), jnp.int32),
            pltpu.VMEM((_MSG_K,), jnp.int32),
            pltpu.VMEM((_MSG_K,), jnp.float32),
            pltpu.VMEM((_MSG_K, P), jnp.float32),
            pltpu.VMEM_SHARED((NP, P), jnp.float32),
            pltpu.SemaphoreType.DMA,
        ],
    )
    def msg_kernel(hp_hbm, row_hbm, col_hbm, ew_hbm, out0, out1,
                   row_v, col_v, ew_v, msg_v, acc, sem):
        c = lax.axis_index("c")
        s = lax.axis_index("s")
        wid = c * 16 + s
        zvec = jnp.zeros((16,), jnp.float32)

        ZR = 128

        def zbody(j, _):
            for p in range(P // 16):
                msg_v[j, pl.ds(p * 16, 16)] = zvec
            return 0

        lax.fori_loop(0, ZR, zbody, 0)
        rs = s * NPT
        for r in range(NPT // ZR):
            pltpu.sync_copy(msg_v.at[pl.ds(0, ZR)],
                            acc.at[pl.ds(rs + r * ZR, ZR)])
        plsc.subcore_barrier()

        def chunk(ch, _):
            base = wid * ET + ch * _MSG_K
            pltpu.sync_copy(row_hbm.at[pl.ds(base, _MSG_K)], row_v)
            pltpu.sync_copy(col_hbm.at[pl.ds(base, _MSG_K)], col_v)
            pltpu.sync_copy(ew_hbm.at[pl.ds(base, _MSG_K)], ew_v)
            pltpu.async_copy(hp_hbm.at[row_v], msg_v, sem).wait()

            def scale(j, _):
                w = plsc.load_gather(ew_v, [jnp.full((16,), j, jnp.int32)])
                for p in range(P // 16):
                    sl = pl.ds(p * 16, 16)
                    msg_v[j, sl] = msg_v[j, sl] * w
                return 0

            lax.fori_loop(0, _MSG_K, scale, 0)
            pltpu.sync_copy(msg_v, acc.at[col_v], add=True)
            return 0

        lax.fori_loop(0, ET // _MSG_K, chunk, 0)
        plsc.subcore_barrier()

        @pl.when(c == 0)
        def _():
            pltpu.sync_copy(acc.at[pl.ds(rs, NPT)], out0.at[pl.ds(rs, NPT)])

        @pl.when(c == 1)
        def _():
            pltpu.sync_copy(acc.at[pl.ds(rs, NPT)], out1.at[pl.ds(rs, NPT)])

    return msg_kernel


_MSG_KERNELS = {P: _make_msg_kernel(P) for P in sorted(set(PADS))}


def kernel(x, edge_index, edge_attr, pre_w, pre_b,
           c1_lin, c1_w1, c1_b1, c1_w2, c1_b2,
           c2_lin, c2_w1, c2_b1, c2_w2, c2_b2,
           c3_lin, c3_w1, c3_b1, c3_w2, c3_b2,
           c4_lin, c4_w1, c4_b1, c4_w2, c4_b2,
           n1_w, n1_b, g2_w, g2_b, g2_ms, g3_w, g3_b, g3_ms,
           g4_w, g4_b, g4_ms):
    row = edge_index[0]
    col = edge_index[1]

    lins = [c1_lin, c2_lin, c3_lin, c4_lin]
    w1s = jnp.stack([c1_w1, c2_w1, c3_w1, c4_w1])
    b1s = jnp.stack([c1_b1, c2_b1, c3_b1, c4_b1])
    w2s = jnp.stack([c1_w2, c2_w2, c3_w2, c4_w2])
    b2s = jnp.stack([c1_b2, c2_b2, c3_b2, c4_b2])

    ews = _edge_mlp(edge_attr, w1s, b1s, w2s, b2s)
    ew_list = [ews[:, l] for l in range(4)]
    degs = _deg_kernel(col, *ew_list)

    in_pads = [DIMS[0]] + PADS[:3]
    lin_pads = [
        jnp.pad(lins[l], ((0, in_pads[l] - DIMS[l]), (0, PADS[l] - DIMS[l + 1])))
        for l in range(4)
    ]

    def padv(v, P):
        return jnp.pad(v, (0, P - v.shape[0]))

    norm_w = [padv(n1_w, PADS[0]), padv(g2_w, PADS[1]), padv(g3_w, PADS[2])]
    norm_b = [padv(n1_b, PADS[0]), padv(g2_b, PADS[1]), padv(g3_b, PADS[2])]
    norm_ms = [jnp.zeros((PADS[0],), jnp.float32), padv(g2_ms, PADS[1]),
               padv(g3_ms, PADS[2])]

    hp, dis = _pre1(x, pre_w, pre_b, lin_pads[0], degs[0], degs[4])
    for l in range(3):
        agg0, agg1 = _MSG_KERNELS[PADS[l]](hp, row, col, ew_list[l])
        hp, dis = _mid(l, agg0, agg1, hp, dis, norm_w[l], norm_b[l],
                       norm_ms[l], lin_pads[l + 1], degs[l + 1], degs[l + 5])
    agg0, agg1 = _MSG_KERNELS[PADS[3]](hp, row, col, ew_list[3])
    return _final(agg0, agg1, hp, dis, g4_w, g4_b, g4_ms)
